# Initial kernel scaffold; baseline (speedup 1.0000x reference)
import jax
import jax.numpy as jnp
from jax.experimental import pallas as pl


def kernel(x, edge_index, Wr0, Wn0, b0, Wr1, Wn1, b1, Wr2, Wn2, b2, Wr3, Wn3, b3, Wr4, Wn4, b4):
    raise NotImplementedError("write your pallas kernel here")


# speedup vs baseline: 10.0825x; 10.0825x over previous
import functools

import jax
import jax.numpy as jnp
from jax import lax
from jax.experimental import pallas as pl
from jax.experimental.pallas import tpu as pltpu
from jax.experimental.pallas import tpu_sc as plsc

_N = 10000
_E = 160000
_IN = 128
_CH = 32

_NC, _NS, _L = 2, 16, 16
_NW = _NC * _NS
_K = 128
_EP = 163840
_EW = _EP // _NW
_NCH = _EW // _K
_NP = 10240
_RPT = _NP // _NS


def _seg_sum_body(fo, xn_hbm, src_hbm, dst_hbm, out_hbm,
                  src_v, dst_v, rows_v, zrow, acc_sh, sem):
    c = lax.axis_index("c")
    s = lax.axis_index("s")
    wid = s * _NC + c

    zv = jnp.zeros((_L,), jnp.float32)
    for r in range(_L):
        for q in range(fo // _L):
            zrow[r, pl.ds(q * _L, _L)] = zv

    def _zero(i, carry):
        pltpu.sync_copy(zrow, acc_sh.at[pl.ds(s * _RPT + i * _L, _L)])
        return carry
    lax.fori_loop(0, _RPT // _L, _zero, 0)
    plsc.subcore_barrier()

    pltpu.sync_copy(src_hbm.at[wid], src_v)
    pltpu.sync_copy(dst_hbm.at[wid], dst_v)

    def _chunk(j, carry):
        pltpu.async_copy(xn_hbm.at[src_v.at[j]], rows_v, sem).wait()
        pltpu.sync_copy(rows_v, acc_sh.at[dst_v.at[j]], add=True)
        return carry
    lax.fori_loop(0, _NCH, _chunk, 0)
    plsc.subcore_barrier()

    pltpu.sync_copy(acc_sh.at[pl.ds(s * _RPT, _RPT)],
                    out_hbm.at[c, pl.ds(s * _RPT, _RPT)])


@functools.cache
def _seg_sum(fo):
    mesh = plsc.VectorSubcoreMesh(core_axis_name="c", subcore_axis_name="s",
                                  num_cores=_NC, num_subcores=_NS)
    return pl.kernel(
        functools.partial(_seg_sum_body, fo),
        out_type=jax.ShapeDtypeStruct((_NC, _NP, fo), jnp.float32),
        mesh=mesh,
        compiler_params=pltpu.CompilerParams(use_tc_tiling_on_sc=False),
        scratch_types=[
            pltpu.VMEM((_NCH, _K), jnp.int32),
            pltpu.VMEM((_NCH, _K), jnp.int32),
            pltpu.VMEM((_K, fo), jnp.float32),
            pltpu.VMEM((_L, fo), jnp.float32),
            pltpu.VMEM_SHARED((_NP, fo), jnp.float32),
            pltpu.SemaphoreType.DMA,
        ],
    )


_BM = 1000


def _mm_body(fo, h_ref, w_ref, b_ref, on_ref, or_ref):
    t = jnp.dot(h_ref[...], w_ref[...], preferred_element_type=jnp.float32)
    on_ref[...] = t[:, :fo]
    or_ref[...] = t[:, fo:] + b_ref[...]


def _mm(h, wcat, b):
    fi = h.shape[1]
    fo = wcat.shape[1] // 2
    return pl.pallas_call(
        functools.partial(_mm_body, fo),
        grid=(_N // _BM,),
        in_specs=[
            pl.BlockSpec((_BM, fi), lambda i: (i, 0)),
            pl.BlockSpec((fi, 2 * fo), lambda i: (0, 0)),
            pl.BlockSpec((1, fo), lambda i: (0, 0)),
        ],
        out_specs=[
            pl.BlockSpec((_BM, fo), lambda i: (i, 0)),
            pl.BlockSpec((_BM, fo), lambda i: (i, 0)),
        ],
        out_shape=[
            jax.ShapeDtypeStruct((_N, fo), jnp.float32),
            jax.ShapeDtypeStruct((_N, fo), jnp.float32),
        ],
    )(h, wcat, b.reshape(1, fo))


def _comb_body(apply_lr, xr_ref, a0_ref, a1_ref, o_ref):
    t = xr_ref[...] + a0_ref[0] + a1_ref[0]
    if apply_lr:
        t = jnp.where(t > 0, t, 0.01 * t)
    o_ref[...] = t


def _comb(xr, agg, apply_lr):
    fo = xr.shape[1]
    return pl.pallas_call(
        functools.partial(_comb_body, apply_lr),
        grid=(_N // _BM,),
        in_specs=[
            pl.BlockSpec((_BM, fo), lambda i: (i, 0)),
            pl.BlockSpec((1, _BM, fo), lambda i: (0, i, 0)),
            pl.BlockSpec((1, _BM, fo), lambda i: (1, i, 0)),
        ],
        out_specs=pl.BlockSpec((_BM, fo), lambda i: (i, 0)),
        out_shape=jax.ShapeDtypeStruct((_N, fo), jnp.float32),
    )(xr, agg, agg)


_BR = 1024


def _recon_body(a_ref, b_ref, o_ref):
    t = jnp.dot(a_ref[...], b_ref[...].T, preferred_element_type=jnp.float32)
    o_ref[...] = jax.nn.sigmoid(t)


def _recon(re):
    g = pl.cdiv(_N, _BR)
    return pl.pallas_call(
        _recon_body,
        grid=(g, g),
        in_specs=[
            pl.BlockSpec((_BR, _CH), lambda i, j: (i, 0)),
            pl.BlockSpec((_BR, _CH), lambda i, j: (j, 0)),
        ],
        out_specs=pl.BlockSpec((_BR, _BR), lambda i, j: (i, j)),
        out_shape=jax.ShapeDtypeStruct((_N, _N), jnp.float32),
    )(re, re)


def _conv(h, src_r, dst_r, wcat, b, apply_lr):
    xn, xr = _mm(h, wcat, b)
    agg = _seg_sum(xn.shape[1])(xn, src_r, dst_r)
    return _comb(xr, agg, apply_lr)


def kernel(x, edge_index, Wr0, Wn0, b0, Wr1, Wn1, b1, Wr2, Wn2, b2,
           Wr3, Wn3, b3, Wr4, Wn4, b4):
    xb = x[0]
    ei = edge_index[0]
    src = ei[:, 0]
    dst = ei[:, 1]
    pad = _EP - _E
    src_r = jnp.concatenate([src, jnp.zeros((pad,), jnp.int32)]).reshape(
        _NW, _NCH, _K)
    dst_r = jnp.concatenate([dst, jnp.full((pad,), _N, jnp.int32)]).reshape(
        _NW, _NCH, _K)

    wc = [jnp.concatenate([wn, wr], axis=1)
          for wn, wr in ((Wn0, Wr0), (Wn1, Wr1), (Wn2, Wr2), (Wn3, Wr3),
                         (Wn4, Wr4))]

    h = _conv(xb, src_r, dst_r, wc[0], b0, True)
    z = _conv(h, src_r, dst_r, wc[1], b1, True)
    re = _conv(z, src_r, dst_r, wc[2], b2, True)
    recon = _recon(re)
    xd = _conv(z, src_r, dst_r, wc[3], b3, True)
    xo = _conv(xd, src_r, dst_r, wc[4], b4, False)

    return (recon.reshape(1, _N, _N), xo.reshape(1, _N, _IN),
            z.reshape(1, _N, 2 * _CH))


# speedup vs baseline: 11.1442x; 1.1053x over previous
import functools

import jax
import jax.numpy as jnp
from jax import lax
from jax.experimental import pallas as pl
from jax.experimental.pallas import tpu as pltpu
from jax.experimental.pallas import tpu_sc as plsc

_N = 10000
_E = 160000
_IN = 128
_CH = 32

_NC, _NS, _L = 2, 16, 16
_NW = _NC * _NS
_K = 128
_EP = 163840
_EW = _EP // _NW
_NCH = _EW // _K
_NP = 10240
_RPT = _NP // _NS


def _seg_sum_body(fo, xn_hbm, src_hbm, dst_hbm, out_hbm,
                  src_v, dst_v, rows_a, rows_b, zrow, acc_sh, sem_a, sem_b):
    c = lax.axis_index("c")
    s = lax.axis_index("s")
    wid = s * _NC + c

    zv = jnp.zeros((_L,), jnp.float32)
    for r in range(_L):
        for q in range(fo // _L):
            zrow[r, pl.ds(q * _L, _L)] = zv

    def _zero(i, carry):
        pltpu.sync_copy(zrow, acc_sh.at[pl.ds(s * _RPT + i * _L, _L)])
        return carry
    lax.fori_loop(0, _RPT // _L, _zero, 0)
    plsc.subcore_barrier()

    pltpu.sync_copy(src_hbm.at[wid], src_v)
    pltpu.sync_copy(dst_hbm.at[wid], dst_v)

    bufs = (rows_a, rows_b)
    sems = (sem_a, sem_b)
    pltpu.async_copy(xn_hbm.at[src_v.at[0]], rows_a, sem_a)
    pltpu.async_copy(xn_hbm.at[src_v.at[1]], rows_b, sem_b)

    def _pair(jj, carry):
        j = jj * 2
        for p in range(2):
            pltpu.make_async_copy(xn_hbm.at[src_v.at[j + p]],
                                  bufs[p], sems[p]).wait()
            pltpu.sync_copy(bufs[p], acc_sh.at[dst_v.at[j + p]], add=True)

            @pl.when(j + p + 2 < _NCH)
            def _():
                pltpu.async_copy(xn_hbm.at[src_v.at[j + p + 2]],
                                 bufs[p], sems[p])
        return carry
    lax.fori_loop(0, _NCH // 2, _pair, 0)
    plsc.subcore_barrier()

    pltpu.sync_copy(acc_sh.at[pl.ds(s * _RPT, _RPT)],
                    out_hbm.at[c, pl.ds(s * _RPT, _RPT)])


@functools.cache
def _seg_sum(fo):
    mesh = plsc.VectorSubcoreMesh(core_axis_name="c", subcore_axis_name="s",
                                  num_cores=_NC, num_subcores=_NS)
    return pl.kernel(
        functools.partial(_seg_sum_body, fo),
        out_type=jax.ShapeDtypeStruct((_NC, _NP, fo), jnp.float32),
        mesh=mesh,
        compiler_params=pltpu.CompilerParams(use_tc_tiling_on_sc=False),
        scratch_types=[
            pltpu.VMEM((_NCH, _K), jnp.int32),
            pltpu.VMEM((_NCH, _K), jnp.int32),
            pltpu.VMEM((_K, fo), jnp.float32),
            pltpu.VMEM((_K, fo), jnp.float32),
            pltpu.VMEM((_L, fo), jnp.float32),
            pltpu.VMEM_SHARED((_NP, fo), jnp.float32),
            pltpu.SemaphoreType.DMA,
            pltpu.SemaphoreType.DMA,
        ],
    )


_BM = 1000


def _mm_body(fo, h_ref, w_ref, b_ref, on_ref, or_ref):
    t = jnp.dot(h_ref[...], w_ref[...], preferred_element_type=jnp.float32)
    on_ref[...] = t[:, :fo]
    or_ref[...] = t[:, fo:] + b_ref[...]


def _mm(h, wcat, b):
    fi = h.shape[1]
    fo = wcat.shape[1] // 2
    return pl.pallas_call(
        functools.partial(_mm_body, fo),
        grid=(_N // _BM,),
        in_specs=[
            pl.BlockSpec((_BM, fi), lambda i: (i, 0)),
            pl.BlockSpec((fi, 2 * fo), lambda i: (0, 0)),
            pl.BlockSpec((1, fo), lambda i: (0, 0)),
        ],
        out_specs=[
            pl.BlockSpec((_BM, fo), lambda i: (i, 0)),
            pl.BlockSpec((_BM, fo), lambda i: (i, 0)),
        ],
        out_shape=[
            jax.ShapeDtypeStruct((_N, fo), jnp.float32),
            jax.ShapeDtypeStruct((_N, fo), jnp.float32),
        ],
    )(h, wcat, b.reshape(1, fo))


def _comb_body(apply_lr, xr_ref, a0_ref, a1_ref, o_ref):
    t = xr_ref[...] + a0_ref[0] + a1_ref[0]
    if apply_lr:
        t = jnp.where(t > 0, t, 0.01 * t)
    o_ref[...] = t


def _comb(xr, agg, apply_lr):
    fo = xr.shape[1]
    return pl.pallas_call(
        functools.partial(_comb_body, apply_lr),
        grid=(_N // _BM,),
        in_specs=[
            pl.BlockSpec((_BM, fo), lambda i: (i, 0)),
            pl.BlockSpec((1, _BM, fo), lambda i: (0, i, 0)),
            pl.BlockSpec((1, _BM, fo), lambda i: (1, i, 0)),
        ],
        out_specs=pl.BlockSpec((_BM, fo), lambda i: (i, 0)),
        out_shape=jax.ShapeDtypeStruct((_N, fo), jnp.float32),
    )(xr, agg, agg)


_BR = 1024


def _recon_body(a_ref, b_ref, o_ref):
    t = jnp.dot(a_ref[...], b_ref[...].T, preferred_element_type=jnp.float32)
    o_ref[...] = jax.nn.sigmoid(t)


def _recon(re):
    g = pl.cdiv(_N, _BR)
    return pl.pallas_call(
        _recon_body,
        grid=(g, g),
        in_specs=[
            pl.BlockSpec((_BR, _CH), lambda i, j: (i, 0)),
            pl.BlockSpec((_BR, _CH), lambda i, j: (j, 0)),
        ],
        out_specs=pl.BlockSpec((_BR, _BR), lambda i, j: (i, j)),
        out_shape=jax.ShapeDtypeStruct((_N, _N), jnp.float32),
    )(re, re)


def _conv(h, src_r, dst_r, wcat, b, apply_lr):
    xn, xr = _mm(h, wcat, b)
    agg = _seg_sum(xn.shape[1])(xn, src_r, dst_r)
    return _comb(xr, agg, apply_lr)


def kernel(x, edge_index, Wr0, Wn0, b0, Wr1, Wn1, b1, Wr2, Wn2, b2,
           Wr3, Wn3, b3, Wr4, Wn4, b4):
    xb = x[0]
    ei = edge_index[0]
    src = ei[:, 0]
    dst = ei[:, 1]
    pad = _EP - _E
    src_r = jnp.concatenate([src, jnp.zeros((pad,), jnp.int32)]).reshape(
        _NW, _NCH, _K)
    dst_r = jnp.concatenate([dst, jnp.full((pad,), _N, jnp.int32)]).reshape(
        _NW, _NCH, _K)

    wc = [jnp.concatenate([wn, wr], axis=1)
          for wn, wr in ((Wn0, Wr0), (Wn1, Wr1), (Wn2, Wr2), (Wn3, Wr3),
                         (Wn4, Wr4))]

    h = _conv(xb, src_r, dst_r, wc[0], b0, True)
    z = _conv(h, src_r, dst_r, wc[1], b1, True)
    re = _conv(z, src_r, dst_r, wc[2], b2, True)
    recon = _recon(re)
    xd = _conv(z, src_r, dst_r, wc[3], b3, True)
    xo = _conv(xd, src_r, dst_r, wc[4], b4, False)

    return (recon.reshape(1, _N, _N), xo.reshape(1, _N, _IN),
            z.reshape(1, _N, 2 * _CH))
